# Initial kernel scaffold; baseline (speedup 1.0000x reference)
import jax
import jax.numpy as jnp
from jax.experimental import pallas as pl


def kernel(x, table):
    raise NotImplementedError("write your pallas kernel here")


# speedup vs baseline: 1.2055x; 1.2055x over previous
import functools
import math

import jax
import jax.numpy as jnp
from jax import lax
from jax.experimental import pallas as pl
from jax.experimental.pallas import tpu as pltpu
from jax.experimental.pallas import tpu_sc as plsc

_EMB = 512
_SCALE = math.sqrt(_EMB)
_LANES = 16

_NC = 2
_NS = 16
_NW = _NC * _NS

_B = 4096 * 50
_BPW = _B // _NW
_C = 40
_NCHUNK = _BPW // _C
_NBUF = 4
_OUTER = _NCHUNK // _NBUF


def _make_sc_kernel():
  mesh = plsc.VectorSubcoreMesh(core_axis_name="c", subcore_axis_name="s")

  @functools.partial(
      pl.kernel,
      out_type=jax.ShapeDtypeStruct((_B, _EMB), jnp.float32),
      mesh=mesh,
      scratch_types=(
          [pltpu.VMEM((_BPW,), jnp.int32),
           pltpu.VMEM((_NBUF, _C, _EMB), jnp.float32)]
          + [pltpu.SemaphoreType.DMA] * (2 * _NBUF)
      ),
  )
  def sc_embed(idx_hbm, table_hbm, out_hbm, idx_v, buf, *sems):
    g_sems = sems[:_NBUF]
    o_sems = sems[_NBUF:]
    wid = lax.axis_index("s") * _NC + lax.axis_index("c")
    base = wid * _BPW
    pltpu.sync_copy(idx_hbm.at[pl.ds(base, _BPW)], idx_v)

    def gather_copy(i, b):
      return pltpu.make_async_copy(
          table_hbm.at[idx_v.at[pl.ds(i * _C, _C)]], buf.at[b], g_sems[b])

    def out_copy(i, b):
      return pltpu.make_async_copy(
          buf.at[b], out_hbm.at[pl.ds(base + i * _C, _C)], o_sems[b])

    gather_copy(0, 0).start()
    gather_copy(1, 1).start()

    def outer(j, carry):
      for b in range(_NBUF):
        i = j * _NBUF + b
        gather_copy(i, b).wait()

        def scale_row(r, c2, _b=b):
          for c in range(_EMB // _LANES):
            buf[_b, r, pl.ds(c * _LANES, _LANES)] = (
                buf[_b, r, pl.ds(c * _LANES, _LANES)] * _SCALE)
          return c2
        lax.fori_loop(0, _C, scale_row, 0)

        out_copy(i, b).start()

        bn = (b + 2) % _NBUF
        if b < 2:
          @pl.when(j >= 1)
          def _(i=i, b=b, bn=bn):
            out_copy(i - 2, bn).wait()
          gather_copy(i + 2, bn).start()
        else:
          @pl.when(j < _OUTER - 1)
          def _(i=i, b=b, bn=bn):
            out_copy(i - 2, bn).wait()
            gather_copy(i + 2, bn).start()
      return carry

    lax.fori_loop(0, _OUTER, outer, 0)

    for b in range(_NBUF):
      out_copy(_NCHUNK - _NBUF + b, b).wait()

  return sc_embed


_SC_EMBED = _make_sc_kernel()


def kernel(x, table):
  idx_flat = x.reshape(-1)
  out = _SC_EMBED(idx_flat, table)
  return out.reshape(x.shape + (table.shape[1],))
